# baseline (speedup 1.0000x reference)
import jax
import jax.numpy as jnp
import numpy as np
from jax.experimental import pallas as pl
from jax.experimental.pallas import tpu as pltpu

D = 2048
H3 = 192
BR = 512


def _fused_kernel(obs_ref, w0_ref, b0_ref, w1_ref, b1_ref, w2_ref, b2_ref,
                  pi_ref, v_ref):
    x = obs_ref[:, :D]
    gate = obs_ref[:, D:D + 3]
    h = jnp.tanh(jnp.dot(x, w0_ref[:, :], preferred_element_type=jnp.float32)
                 + b0_ref[:, :])
    h = jnp.tanh(jnp.dot(h, w1_ref[:, :], preferred_element_type=jnp.float32)
                 + b1_ref[:, :])
    o = jnp.dot(h, w2_ref[:, :], preferred_element_type=jnp.float32) + b2_ref[:, :]
    lf = o[:, 0:32]
    lc = o[:, 32:64]
    pf = jax.nn.softmax(lf, axis=-1)
    pc = jax.nn.softmax(lc, axis=-1)
    mask = jnp.all(jnp.abs(gate) <= 0.1, axis=-1, keepdims=True)
    zero = jnp.zeros_like(pf)
    pi_ref[:, :] = jnp.concatenate(
        [jnp.where(mask, pf, zero), jnp.where(mask, zero, pc)], axis=-1)
    v_ref[:, :] = o[:, 64:65]


def kernel(observation, prev_action, prev_reward,
           pf_W0, pf_b0, pf_W1, pf_b1, pf_W2, pf_b2,
           pc_W0, pc_b0, pc_W1, pc_b1, pc_W2, pc_b2,
           v_W0, v_b0, v_W1, v_b1, v_W2, v_b2):
    B = observation.shape[0]
    f32 = jnp.float32

    W0 = jnp.concatenate([pf_W0, pc_W0, v_W0], axis=1)
    b0 = jnp.concatenate([pf_b0, pc_b0, v_b0])[None, :]
    W1 = jax.scipy.linalg.block_diag(pf_W1, pc_W1, v_W1)
    b1 = jnp.concatenate([pf_b1, pc_b1, v_b1])[None, :]
    W2 = jnp.zeros((H3, 128), dtype=f32)
    W2 = W2.at[0:64, 0:32].set(pf_W2)
    W2 = W2.at[64:128, 32:64].set(pc_W2)
    W2 = W2.at[128:192, 64:65].set(v_W2)
    b2 = jnp.zeros((128,), dtype=f32)
    b2 = b2.at[0:32].set(pf_b2)
    b2 = b2.at[32:64].set(pc_b2)
    b2 = b2.at[64].set(v_b2[0])
    b2 = b2[None, :]

    grid = (B // BR,)
    rep = lambda i: (0, 0)
    pi, v = pl.pallas_call(
        _fused_kernel,
        grid=grid,
        in_specs=[
            pl.BlockSpec((BR, D + 3), lambda i: (i, 0)),
            pl.BlockSpec((D, H3), rep),
            pl.BlockSpec((1, H3), rep),
            pl.BlockSpec((H3, H3), rep),
            pl.BlockSpec((1, H3), rep),
            pl.BlockSpec((H3, 128), rep),
            pl.BlockSpec((1, 128), rep),
        ],
        out_specs=[
            pl.BlockSpec((BR, 64), lambda i: (i, 0)),
            pl.BlockSpec((BR, 1), lambda i: (i, 0)),
        ],
        out_shape=[
            jax.ShapeDtypeStruct((B, 64), f32),
            jax.ShapeDtypeStruct((B, 1), f32),
        ],
        compiler_params=pltpu.CompilerParams(
            dimension_semantics=("parallel",)),
    )(observation, W0, b0, W1, b1, W2, b2)
    return (pi, v[:, 0])


# speedup vs baseline: 1.1460x; 1.1460x over previous
import jax
import jax.numpy as jnp
import numpy as np
from jax.experimental import pallas as pl
from jax.experimental.pallas import tpu as pltpu

D = 2048
H3 = 192
BR = 512


def _fused_kernel(obs_ref, w0_ref, b0_ref, w1_ref, b1_ref, w2_ref, b2_ref,
                  seg_ref, pi_ref, v_ref):
    x = obs_ref[:, :D].astype(jnp.bfloat16)
    gate = obs_ref[:, D:D + 3]
    h = jnp.tanh(jnp.dot(x, w0_ref[:, :], preferred_element_type=jnp.float32)
                 + b0_ref[:, :])
    h = jnp.tanh(jnp.dot(h, w1_ref[:, :], preferred_element_type=jnp.float32)
                 + b1_ref[:, :])
    o = jnp.dot(h, w2_ref[:, :], preferred_element_type=jnp.float32) + b2_ref[:, :]
    e = jnp.exp(o[:, 0:64])
    s = jnp.dot(e, seg_ref[:, :], preferred_element_type=jnp.float32)
    p = e / s
    mask = jnp.all(jnp.abs(gate) <= 0.1, axis=-1, keepdims=True)
    col = jax.lax.broadcasted_iota(jnp.int32, (1, 64), 1) < 32
    pi_ref[:, :] = jnp.where(mask == col, p, 0.0)
    v_ref[:, :] = o[:, 64:65]


def kernel(observation, prev_action, prev_reward,
           pf_W0, pf_b0, pf_W1, pf_b1, pf_W2, pf_b2,
           pc_W0, pc_b0, pc_W1, pc_b1, pc_W2, pc_b2,
           v_W0, v_b0, v_W1, v_b1, v_W2, v_b2):
    B = observation.shape[0]
    f32 = jnp.float32

    W0 = jnp.concatenate([pf_W0, pc_W0, v_W0], axis=1).astype(jnp.bfloat16)
    b0 = jnp.concatenate([pf_b0, pc_b0, v_b0])[None, :]
    W1 = jax.scipy.linalg.block_diag(pf_W1, pc_W1, v_W1)
    b1 = jnp.concatenate([pf_b1, pc_b1, v_b1])[None, :]
    W2 = jnp.zeros((H3, 128), dtype=f32)
    W2 = W2.at[0:64, 0:32].set(pf_W2)
    W2 = W2.at[64:128, 32:64].set(pc_W2)
    W2 = W2.at[128:192, 64:65].set(v_W2)
    b2 = jnp.zeros((128,), dtype=f32)
    b2 = b2.at[0:32].set(pf_b2)
    b2 = b2.at[32:64].set(pc_b2)
    b2 = b2.at[64].set(v_b2[0])
    b2 = b2[None, :]
    seg = jax.scipy.linalg.block_diag(jnp.ones((32, 32), f32),
                                      jnp.ones((32, 32), f32))

    grid = (B // BR,)
    rep = lambda i: (0, 0)
    pi, v = pl.pallas_call(
        _fused_kernel,
        grid=grid,
        in_specs=[
            pl.BlockSpec((BR, D + 3), lambda i: (i, 0)),
            pl.BlockSpec((D, H3), rep),
            pl.BlockSpec((1, H3), rep),
            pl.BlockSpec((H3, H3), rep),
            pl.BlockSpec((1, H3), rep),
            pl.BlockSpec((H3, 128), rep),
            pl.BlockSpec((1, 128), rep),
            pl.BlockSpec((64, 64), rep),
        ],
        out_specs=[
            pl.BlockSpec((BR, 64), lambda i: (i, 0)),
            pl.BlockSpec((BR, 1), lambda i: (i, 0)),
        ],
        out_shape=[
            jax.ShapeDtypeStruct((B, 64), f32),
            jax.ShapeDtypeStruct((B, 1), f32),
        ],
        compiler_params=pltpu.CompilerParams(
            dimension_semantics=("parallel",)),
    )(observation, W0, b0, W1, b1, W2, b2, seg)
    return (pi, v[:, 0])


# speedup vs baseline: 1.1775x; 1.0275x over previous
import jax
import jax.numpy as jnp
import numpy as np
from jax.experimental import pallas as pl
from jax.experimental.pallas import tpu as pltpu

D = 2048
H3 = 192
BR = 1024


def _fused_kernel(obs_ref, w0_ref, b0_ref, w1_ref, b1_ref, w2_ref, b2_ref,
                  seg_ref, pi_ref, v_ref):
    x = obs_ref[:, :D].astype(jnp.bfloat16)
    gate = obs_ref[:, D:D + 3]
    h = jnp.tanh(jnp.dot(x, w0_ref[:, :], preferred_element_type=jnp.float32)
                 + b0_ref[:, :])
    h = jnp.tanh(jnp.dot(h, w1_ref[:, :], preferred_element_type=jnp.float32)
                 + b1_ref[:, :])
    o = jnp.dot(h, w2_ref[:, :], preferred_element_type=jnp.float32) + b2_ref[:, :]
    e = jnp.exp(o[:, 0:64])
    s = jnp.dot(e, seg_ref[:, :], preferred_element_type=jnp.float32)
    p = e / s
    mask = jnp.all(jnp.abs(gate) <= 0.1, axis=-1, keepdims=True)
    col = jax.lax.broadcasted_iota(jnp.int32, (1, 64), 1) < 32
    pi_ref[:, :] = jnp.where(mask == col, p, 0.0)
    v_ref[:, :] = o[:, 64:65]


def kernel(observation, prev_action, prev_reward,
           pf_W0, pf_b0, pf_W1, pf_b1, pf_W2, pf_b2,
           pc_W0, pc_b0, pc_W1, pc_b1, pc_W2, pc_b2,
           v_W0, v_b0, v_W1, v_b1, v_W2, v_b2):
    B = observation.shape[0]
    f32 = jnp.float32

    W0 = jnp.concatenate([pf_W0, pc_W0, v_W0], axis=1).astype(jnp.bfloat16)
    b0 = jnp.concatenate([pf_b0, pc_b0, v_b0])[None, :]
    W1 = jax.scipy.linalg.block_diag(pf_W1, pc_W1, v_W1)
    b1 = jnp.concatenate([pf_b1, pc_b1, v_b1])[None, :]
    W2 = jnp.zeros((H3, 128), dtype=f32)
    W2 = W2.at[0:64, 0:32].set(pf_W2)
    W2 = W2.at[64:128, 32:64].set(pc_W2)
    W2 = W2.at[128:192, 64:65].set(v_W2)
    b2 = jnp.zeros((128,), dtype=f32)
    b2 = b2.at[0:32].set(pf_b2)
    b2 = b2.at[32:64].set(pc_b2)
    b2 = b2.at[64].set(v_b2[0])
    b2 = b2[None, :]
    seg = jax.scipy.linalg.block_diag(jnp.ones((32, 32), f32),
                                      jnp.ones((32, 32), f32))

    grid = (B // BR,)
    rep = lambda i: (0, 0)
    pi, v = pl.pallas_call(
        _fused_kernel,
        grid=grid,
        in_specs=[
            pl.BlockSpec((BR, D + 3), lambda i: (i, 0)),
            pl.BlockSpec((D, H3), rep),
            pl.BlockSpec((1, H3), rep),
            pl.BlockSpec((H3, H3), rep),
            pl.BlockSpec((1, H3), rep),
            pl.BlockSpec((H3, 128), rep),
            pl.BlockSpec((1, 128), rep),
            pl.BlockSpec((64, 64), rep),
        ],
        out_specs=[
            pl.BlockSpec((BR, 64), lambda i: (i, 0)),
            pl.BlockSpec((BR, 1), lambda i: (i, 0)),
        ],
        out_shape=[
            jax.ShapeDtypeStruct((B, 64), f32),
            jax.ShapeDtypeStruct((B, 1), f32),
        ],
        compiler_params=pltpu.CompilerParams(
            dimension_semantics=("parallel",)),
    )(observation, W0, b0, W1, b1, W2, b2, seg)
    return (pi, v[:, 0])
